# Initial kernel scaffold; baseline (speedup 1.0000x reference)
import jax
import jax.numpy as jnp
from jax.experimental import pallas as pl


def kernel(word_embed_matrix, target_mask_list, graph_edge_list, W1, b1, Wc0, att_src0, att_dst0, bc0, Wc1, att_src1, att_dst1, bc1, W3, b3):
    raise NotImplementedError("write your pallas kernel here")


# speedup vs baseline: 44.7388x; 44.7388x over previous
import functools

import jax
import jax.numpy as jnp
from jax import lax
from jax.experimental import pallas as pl
from jax.experimental.pallas import tpu as pltpu
from jax.experimental.pallas import tpu_sc as plsc

N_NODES = 10000
N_EDGES = 320000
D_IN = 128
HIDDEN = 128
HEADS = 8
OUTC = HIDDEN // HEADS

NC = 2
NS = 16
NW = NC * NS
EPW = N_EDGES // NW
CHUNK = 80
NCHUNK = EPW // CHUNK
NPAD = 10240
ROWS_PER_TILE = NPAD // NS
ZROWS = 128


def _leaky(t):
    return jnp.where(t >= 0, t, 0.2 * t)


def _elu(x):
    return jnp.where(x > 0, x, jnp.exp(jnp.minimum(x, 0.0)) - 1.0)


def _in_body(emb, w1t, b1, x_o):
    x = jnp.dot(emb[...], w1t[...], preferred_element_type=jnp.float32,
                precision=lax.Precision.HIGHEST)
    x_o[...] = x + b1[...]


def _proj_body(x, wc, ps, pd, h_o, as_o, ad_o, am_o):
    h = jnp.dot(x[...], wc[...], preferred_element_type=jnp.float32,
                precision=lax.Precision.HIGHEST)
    a_s = jnp.dot(h, ps[...], preferred_element_type=jnp.float32,
                precision=lax.Precision.HIGHEST)
    a_d = jnp.dot(h, pd[...], preferred_element_type=jnp.float32,
                precision=lax.Precision.HIGHEST)
    m = jnp.max(a_s, axis=0, keepdims=True)
    h_o[...] = h
    as_o[...] = a_s
    ad_o[...] = a_d
    am_o[...] = _leaky(m + a_d)


def _norm_body(sp, dp, r, bc, x_o):
    sarr = sp[...]
    darr = dp[...]
    den = jnp.dot(darr[0, :N_NODES] + darr[1, :N_NODES], r[...],
                  preferred_element_type=jnp.float32,
                precision=lax.Precision.HIGHEST)
    x = (sarr[0, :N_NODES] + sarr[1, :N_NODES]) / (den + 1e-16) + bc[...]
    x_o[...] = _elu(x)


def _fin_body(x, w3, b3, sc_o):
    s = jnp.sum(x[...] * w3[...], axis=1, keepdims=True) + b3[...]
    sc_o[...] = jnp.broadcast_to(s, (N_NODES, 16))


_NF = jnp.float32
_in_call = pl.pallas_call(
    _in_body,
    out_shape=[jax.ShapeDtypeStruct((N_NODES, HIDDEN), _NF)],
)
_proj_call = pl.pallas_call(
    _proj_body,
    out_shape=[jax.ShapeDtypeStruct((N_NODES, HIDDEN), _NF),
               jax.ShapeDtypeStruct((N_NODES, 16), _NF),
               jax.ShapeDtypeStruct((N_NODES, 16), _NF),
               jax.ShapeDtypeStruct((N_NODES, 16), _NF)],
)
_norm_call = pl.pallas_call(
    _norm_body,
    out_shape=[jax.ShapeDtypeStruct((N_NODES, HIDDEN), _NF)],
)
_fin_call = pl.pallas_call(
    _fin_body,
    out_shape=[jax.ShapeDtypeStruct((N_NODES, 16), _NF)],
)


@functools.partial(
    pl.kernel,
    mesh=plsc.VectorSubcoreMesh(core_axis_name="c", subcore_axis_name="s"),
    compiler_params=pltpu.CompilerParams(use_tc_tiling_on_sc=False),
    out_type=[jax.ShapeDtypeStruct((NC, NPAD, HIDDEN), _NF),
              jax.ShapeDtypeStruct((NC, NPAD, 16), _NF)],
    scratch_types=[
        pltpu.VMEM((1, CHUNK), jnp.int32),
        pltpu.VMEM((1, CHUNK), jnp.int32),
        pltpu.VMEM((CHUNK, 16), _NF),
        pltpu.VMEM((CHUNK, 16), _NF),
        pltpu.VMEM((CHUNK, 16), _NF),
        pltpu.VMEM((CHUNK, HIDDEN), _NF),
        pltpu.VMEM((CHUNK, 16), _NF),
        pltpu.VMEM_SHARED((NPAD, HIDDEN), _NF),
        pltpu.VMEM_SHARED((NPAD, 16), _NF),
    ],
)
def _edge_call(src_h, dst_h, as_h, ad_h, am_h, h_h, out_s, out_d,
               srci, dsti, asv, adv, amv, hv, exv, acc_s, acc_d):
    c = lax.axis_index("c")
    s = lax.axis_index("s")
    wid = s * NC + c
    rbase = s * ROWS_PER_TILE

    def zrow(i, carry):
        for k in range(HIDDEN // 16):
            hv[i, pl.ds(16 * k, 16)] = jnp.zeros((16,), _NF)
        exv[i] = jnp.zeros((16,), _NF)
        return carry

    lax.fori_loop(0, CHUNK, zrow, 0)

    def zcp(p, carry):
        r0 = rbase + p * CHUNK
        pltpu.sync_copy(hv, acc_s.at[pl.ds(r0, CHUNK)])
        pltpu.sync_copy(exv, acc_d.at[pl.ds(r0, CHUNK)])
        return carry

    lax.fori_loop(0, ROWS_PER_TILE // CHUNK, zcp, 0)
    plsc.subcore_barrier()

    mask8 = lax.iota(jnp.int32, 16) < 8

    def chunk(j, carry):
        pltpu.sync_copy(src_h.at[wid, pl.ds(j, 1)], srci)
        pltpu.sync_copy(dst_h.at[wid, pl.ds(j, 1)], dsti)
        pltpu.sync_copy(as_h.at[srci.at[0]], asv)
        pltpu.sync_copy(ad_h.at[dsti.at[0]], adv)
        pltpu.sync_copy(am_h.at[dsti.at[0]], amv)
        pltpu.sync_copy(h_h.at[srci.at[0]], hv)

        def edge(e, ecarry):
            a = asv[e] + adv[e]
            ex = jnp.exp(_leaky(a) - amv[e])
            exm = jnp.where(mask8, ex, 0.0)
            exv[e] = exm
            for k in range(HEADS):
                w = exm[k]
                hv[e, pl.ds(16 * k, 16)] = hv[e, pl.ds(16 * k, 16)] * w
            return ecarry

        lax.fori_loop(0, CHUNK, edge, 0)
        pltpu.sync_copy(exv, acc_d.at[dsti.at[0]], add=True)
        pltpu.sync_copy(hv, acc_s.at[dsti.at[0]], add=True)
        return carry

    lax.fori_loop(0, NCHUNK, chunk, 0)
    plsc.subcore_barrier()

    def drain(p, carry):
        r0 = rbase + p * CHUNK
        pltpu.sync_copy(acc_s.at[pl.ds(r0, CHUNK)], hv)
        pltpu.sync_copy(hv, out_s.at[c, pl.ds(r0, CHUNK)])
        pltpu.sync_copy(acc_d.at[pl.ds(r0, CHUNK)], exv)
        pltpu.sync_copy(exv, out_d.at[c, pl.ds(r0, CHUNK)])
        return carry

    lax.fori_loop(0, ROWS_PER_TILE // CHUNK, drain, 0)


@functools.partial(
    pl.kernel,
    mesh=plsc.VectorSubcoreMesh(core_axis_name="c", subcore_axis_name="s"),
    compiler_params=pltpu.CompilerParams(use_tc_tiling_on_sc=False),
    out_type=[jax.ShapeDtypeStruct((256, 16), _NF)],
    scratch_types=[
        pltpu.VMEM((2, 128), jnp.int32),
        pltpu.VMEM((128, 16), _NF),
    ],
)
def _tgt_call(scores_h, tgt_h, out_h, tg_v, ot_v):
    c = lax.axis_index("c")
    s = lax.axis_index("s")

    @pl.when(jnp.logical_and(c == 0, s == 0))
    def _():
        pltpu.sync_copy(tgt_h, tg_v)
        for p in range(2):
            pltpu.sync_copy(scores_h.at[tg_v.at[p]], ot_v)
            pltpu.sync_copy(ot_v, out_h.at[pl.ds(128 * p, 128)])


def _expand_att(att):
    a = att.reshape(HEADS * OUTC).astype(jnp.float32)
    m = jnp.repeat(jnp.eye(HEADS, dtype=jnp.float32), OUTC, axis=0)
    p8 = m * a[:, None]
    return jnp.concatenate([p8, jnp.zeros((HIDDEN, 16 - HEADS), jnp.float32)], axis=1)


def kernel(word_embed_matrix, target_mask_list, graph_edge_list, W1, b1,
           Wc0, att_src0, att_dst0, bc0, Wc1, att_src1, att_dst1, bc1, W3, b3):
    src = graph_edge_list[0].astype(jnp.int32).reshape(NW, NCHUNK, CHUNK)
    dst = graph_edge_list[1].astype(jnp.int32).reshape(NW, NCHUNK, CHUNK)
    r_mat = jnp.repeat(jnp.eye(16, dtype=jnp.float32)[:, :HEADS], OUTC, axis=1)

    (x0,) = _in_call(word_embed_matrix, W1.T, b1.reshape(1, -1))
    wc_s = jnp.stack([Wc0, Wc1])
    ps_s = jnp.stack([_expand_att(att_src0), _expand_att(att_src1)])
    pd_s = jnp.stack([_expand_att(att_dst0), _expand_att(att_dst1)])
    bc_s = jnp.stack([bc0.reshape(1, -1), bc1.reshape(1, -1)])

    def body(x, ws):
        wc, ps, pd, bc = ws
        h, a_s, a_d, am = _proj_call(x, wc, ps, pd)
        s, d = _edge_call(src, dst, a_s, a_d, am, h)
        (xn,) = _norm_call(s, d, r_mat, bc)
        return xn, None

    x2, _ = lax.scan(body, x0, (wc_s, ps_s, pd_s, bc_s))
    (scores,) = _fin_call(x2, W3.reshape(1, -1), b3.reshape(1, 1))
    tgt = target_mask_list.reshape(2, 128).astype(jnp.int32)
    (out2,) = _tgt_call(scores, tgt)
    return out2[:, 0]


# speedup vs baseline: 64.7295x; 1.4468x over previous
import functools

import jax
import jax.numpy as jnp
from jax import lax
from jax.experimental import pallas as pl
from jax.experimental.pallas import tpu as pltpu
from jax.experimental.pallas import tpu_sc as plsc

N_NODES = 10000
N_EDGES = 320000
D_IN = 128
HIDDEN = 128
HEADS = 8
OUTC = HIDDEN // HEADS

NC = 2
NS = 16
NW = NC * NS
EPW = N_EDGES // NW
CHUNK = 80
NCHUNK = EPW // CHUNK
NPAD = 10240
ROWS_PER_TILE = NPAD // NS
ZROWS = 128


def _leaky(t):
    return jnp.where(t >= 0, t, 0.2 * t)


def _elu(x):
    return jnp.where(x > 0, x, jnp.exp(jnp.minimum(x, 0.0)) - 1.0)


def _in_body(emb, w1t, b1, x_o):
    x = jnp.dot(emb[...], w1t[...], preferred_element_type=jnp.float32,
                precision=lax.Precision.HIGHEST)
    x_o[...] = x + b1[...]


def _proj_body(x, wc, ps, pd, h_o, as_o, ad_o, am_o):
    h = jnp.dot(x[...], wc[...], preferred_element_type=jnp.float32,
                precision=lax.Precision.HIGHEST)
    a_s = jnp.dot(h, ps[...], preferred_element_type=jnp.float32,
                precision=lax.Precision.HIGHEST)
    a_d = jnp.dot(h, pd[...], preferred_element_type=jnp.float32,
                precision=lax.Precision.HIGHEST)
    m = jnp.max(a_s, axis=0, keepdims=True)
    h_o[...] = h
    as_o[...] = a_s
    ad_o[...] = a_d
    am_o[...] = _leaky(m + a_d)


def _norm_body(sp, dp, r, bc, x_o):
    sarr = sp[...]
    darr = dp[...]
    den = jnp.dot(darr[0, :N_NODES] + darr[1, :N_NODES], r[...],
                  preferred_element_type=jnp.float32,
                precision=lax.Precision.HIGHEST)
    x = (sarr[0, :N_NODES] + sarr[1, :N_NODES]) / (den + 1e-16) + bc[...]
    x_o[...] = _elu(x)


def _fin_body(x, w3, b3, sc_o):
    s = jnp.sum(x[...] * w3[...], axis=1, keepdims=True) + b3[...]
    sc_o[...] = jnp.broadcast_to(s, (N_NODES, 16))


_NF = jnp.float32
_in_call = pl.pallas_call(
    _in_body,
    out_shape=[jax.ShapeDtypeStruct((N_NODES, HIDDEN), _NF)],
)
_proj_call = pl.pallas_call(
    _proj_body,
    out_shape=[jax.ShapeDtypeStruct((N_NODES, HIDDEN), _NF),
               jax.ShapeDtypeStruct((N_NODES, 16), _NF),
               jax.ShapeDtypeStruct((N_NODES, 16), _NF),
               jax.ShapeDtypeStruct((N_NODES, 16), _NF)],
)
_norm_call = pl.pallas_call(
    _norm_body,
    out_shape=[jax.ShapeDtypeStruct((N_NODES, HIDDEN), _NF)],
)
_fin_call = pl.pallas_call(
    _fin_body,
    out_shape=[jax.ShapeDtypeStruct((N_NODES, 16), _NF)],
)


@functools.partial(
    pl.kernel,
    mesh=plsc.VectorSubcoreMesh(core_axis_name="c", subcore_axis_name="s"),
    compiler_params=pltpu.CompilerParams(use_tc_tiling_on_sc=False),
    out_type=[jax.ShapeDtypeStruct((NC, NPAD, HIDDEN), _NF),
              jax.ShapeDtypeStruct((NC, NPAD, 16), _NF)],
    scratch_types=[
        pltpu.VMEM((2, CHUNK), jnp.int32),
        pltpu.VMEM((CHUNK, 16), _NF),
        pltpu.VMEM((CHUNK, 16), _NF),
        pltpu.VMEM((CHUNK, 16), _NF),
        pltpu.VMEM((CHUNK, HIDDEN), _NF),
        pltpu.VMEM((CHUNK, 16), _NF),
        pltpu.VMEM_SHARED((NPAD, HIDDEN), _NF),
        pltpu.VMEM_SHARED((NPAD, 16), _NF),
        pltpu.SemaphoreType.DMA,
        pltpu.SemaphoreType.DMA,
        pltpu.SemaphoreType.DMA,
        pltpu.SemaphoreType.DMA,
    ],
)
def _edge_call(edge_h, as_h, ad_h, am_h, h_h, out_s, out_d,
               idxv, asv, adv, amv, hv, exv, acc_s, acc_d,
               sem0, sem1, sem2, sem3):
    c = lax.axis_index("c")
    s = lax.axis_index("s")
    wid = s * NC + c
    rbase = s * ROWS_PER_TILE

    def zrow(i, carry):
        for k in range(HIDDEN // 16):
            hv[i, pl.ds(16 * k, 16)] = jnp.zeros((16,), _NF)
        exv[i] = jnp.zeros((16,), _NF)
        return carry

    lax.fori_loop(0, CHUNK, zrow, 0)

    def zcp(p, carry):
        r0 = rbase + p * CHUNK
        pltpu.sync_copy(hv, acc_s.at[pl.ds(r0, CHUNK)])
        pltpu.sync_copy(exv, acc_d.at[pl.ds(r0, CHUNK)])
        return carry

    lax.fori_loop(0, ROWS_PER_TILE // CHUNK, zcp, 0)
    plsc.subcore_barrier()

    mask8 = lax.iota(jnp.int32, 16) < 8

    def chunk(j, carry):
        pltpu.sync_copy(edge_h.at[wid, j], idxv)
        g0 = pltpu.async_copy(as_h.at[idxv.at[0]], asv, sem0)
        g1 = pltpu.async_copy(ad_h.at[idxv.at[1]], adv, sem1)
        g2 = pltpu.async_copy(am_h.at[idxv.at[1]], amv, sem2)
        g3 = pltpu.async_copy(h_h.at[idxv.at[0]], hv, sem3)
        g0.wait()
        g1.wait()
        g2.wait()
        g3.wait()

        def edge(e, ecarry):
            a = asv[e] + adv[e]
            ex = jnp.exp(_leaky(a) - amv[e])
            exm = jnp.where(mask8, ex, 0.0)
            exv[e] = exm
            for k in range(HEADS):
                w = exm[k]
                hv[e, pl.ds(16 * k, 16)] = hv[e, pl.ds(16 * k, 16)] * w
            return ecarry

        lax.fori_loop(0, CHUNK, edge, 0)
        pltpu.sync_copy(exv, acc_d.at[idxv.at[1]], add=True)
        pltpu.sync_copy(hv, acc_s.at[idxv.at[1]], add=True)
        return carry

    lax.fori_loop(0, NCHUNK, chunk, 0)
    plsc.subcore_barrier()

    def drain(p, carry):
        r0 = rbase + p * CHUNK
        pltpu.sync_copy(acc_s.at[pl.ds(r0, CHUNK)], hv)
        pltpu.sync_copy(hv, out_s.at[c, pl.ds(r0, CHUNK)])
        pltpu.sync_copy(acc_d.at[pl.ds(r0, CHUNK)], exv)
        pltpu.sync_copy(exv, out_d.at[c, pl.ds(r0, CHUNK)])
        return carry

    lax.fori_loop(0, ROWS_PER_TILE // CHUNK, drain, 0)


@functools.partial(
    pl.kernel,
    mesh=plsc.VectorSubcoreMesh(core_axis_name="c", subcore_axis_name="s"),
    compiler_params=pltpu.CompilerParams(use_tc_tiling_on_sc=False),
    out_type=[jax.ShapeDtypeStruct((256, 16), _NF)],
    scratch_types=[
        pltpu.VMEM((2, 128), jnp.int32),
        pltpu.VMEM((128, 16), _NF),
    ],
)
def _tgt_call(scores_h, tgt_h, out_h, tg_v, ot_v):
    c = lax.axis_index("c")
    s = lax.axis_index("s")

    @pl.when(jnp.logical_and(c == 0, s == 0))
    def _():
        pltpu.sync_copy(tgt_h, tg_v)
        for p in range(2):
            pltpu.sync_copy(scores_h.at[tg_v.at[p]], ot_v)
            pltpu.sync_copy(ot_v, out_h.at[pl.ds(128 * p, 128)])


def _expand_att(att):
    a = att.reshape(HEADS * OUTC).astype(jnp.float32)
    m = jnp.repeat(jnp.eye(HEADS, dtype=jnp.float32), OUTC, axis=0)
    p8 = m * a[:, None]
    return jnp.concatenate([p8, jnp.zeros((HIDDEN, 16 - HEADS), jnp.float32)], axis=1)


def kernel(word_embed_matrix, target_mask_list, graph_edge_list, W1, b1,
           Wc0, att_src0, att_dst0, bc0, Wc1, att_src1, att_dst1, bc1, W3, b3):
    edges = jnp.swapaxes(
        graph_edge_list.astype(jnp.int32).reshape(2, NW, NCHUNK, CHUNK),
        0, 1).swapaxes(1, 2)
    r_mat = jnp.repeat(jnp.eye(16, dtype=jnp.float32)[:, :HEADS], OUTC, axis=1)

    (x0,) = _in_call(word_embed_matrix, W1.T, b1.reshape(1, -1))
    wc_s = jnp.stack([Wc0, Wc1])
    ps_s = jnp.stack([_expand_att(att_src0), _expand_att(att_src1)])
    pd_s = jnp.stack([_expand_att(att_dst0), _expand_att(att_dst1)])
    bc_s = jnp.stack([bc0.reshape(1, -1), bc1.reshape(1, -1)])

    def body(x, ws):
        wc, ps, pd, bc = ws
        h, a_s, a_d, am = _proj_call(x, wc, ps, pd)
        s, d = _edge_call(edges, a_s, a_d, am, h)
        (xn,) = _norm_call(s, d, r_mat, bc)
        return xn, None

    x2, _ = lax.scan(body, x0, (wc_s, ps_s, pd_s, bc_s))
    (scores,) = _fin_call(x2, W3.reshape(1, -1), b3.reshape(1, 1))
    tgt = target_mask_list.reshape(2, 128).astype(jnp.int32)
    (out2,) = _tgt_call(scores, tgt)
    return out2[:, 0]
